# baseline (speedup 1.0000x reference)
import functools

import jax
import jax.numpy as jnp
from jax import lax
from jax.experimental import pallas as pl
from jax.experimental.pallas import tpu as pltpu
from jax.experimental.pallas import tpu_sc as plsc

MASK_TAU = 0.5
L = 16
NC, NS = 2, 16
NW = NC * NS
BPW = 512
QUARTER = 128


def _sc_body(users_m, items_m, users, pos, neg, out,
             idx_u, idx_p, idx_n, idm_u, idm_p, idm_n,
             bu0, bp0, bn0, bu1, bp1, bn1, x_v, sem0, sem1):
    wid = lax.axis_index("s") * NC + lax.axis_index("c")
    base = wid * BPW
    pltpu.sync_copy(users.at[pl.ds(base, BPW)], idx_u)
    pltpu.sync_copy(pos.at[pl.ds(base, BPW)], idx_p)
    pltpu.sync_copy(neg.at[pl.ds(base, BPW)], idx_n)

    for j in range(BPW // L):
        sl = pl.ds(j * L, L)
        idm_u[sl] = lax.shift_right_logical(idx_u[sl], 1)
        idm_p[sl] = lax.shift_right_logical(idx_p[sl], 1)
        idm_n[sl] = lax.shift_right_logical(idx_n[sl], 1)

    bufs = [(bu0, bp0, bn0, sem0), (bu1, bp1, bn1, sem1)]

    def fire(q):
        bu, bp, bn, sem = bufs[q % 2]
        sl = pl.ds(q * QUARTER, QUARTER)
        return [
            pltpu.async_copy(users_m.at[idm_u.at[sl]], bu, sem),
            pltpu.async_copy(items_m.at[idm_p.at[sl]], bp, sem),
            pltpu.async_copy(items_m.at[idm_n.at[sl]], bn, sem),
        ]

    lanes = lax.iota(jnp.int32, L)
    cb = [jnp.bitwise_and(lanes + k, L - 1) for k in range(L)]

    n_q = BPW // QUARTER
    pend = fire(0)
    for q in range(n_q):
        nxt = fire(q + 1) if q + 1 < n_q else []
        for cp in pend:
            cp.wait()
        pend = nxt
        bu, bp, bn, _ = bufs[q % 2]

        @pl.loop(0, QUARTER // L)
        def _blk(b0):
            esl = pl.ds(q * QUARTER + b0 * L, L)
            par_u = jnp.bitwise_and(idx_u[esl], 1) * 64
            par_p = jnp.bitwise_and(idx_p[esl], 1) * 64
            par_n = jnp.bitwise_and(idx_n[esl], 1) * 64
            row = b0 * L + lanes
            acc = [jnp.zeros((L,), jnp.float32) for _ in range(4)]
            for g in range(4):
                for k in range(L):
                    col = cb[k] + (16 * g)
                    u = plsc.load_gather(bu, [row, par_u + col])
                    p = plsc.load_gather(bp, [row, par_p + col])
                    n = plsc.load_gather(bn, [row, par_n + col])
                    acc[k % 4] = acc[k % 4] + u * (p - n)
            x_v[esl] = (acc[0] + acc[1]) + (acc[2] + acc[3])

    pltpu.sync_copy(x_v, out.at[pl.ds(base, BPW)])


def _sc_scores(users_m2, items_m2, users, pos, neg):
    b = users.shape[0]
    mesh = plsc.VectorSubcoreMesh(core_axis_name="c", subcore_axis_name="s",
                                  num_cores=NC, num_subcores=NS)
    f = pl.kernel(
        _sc_body,
        out_type=jax.ShapeDtypeStruct((b,), jnp.float32),
        mesh=mesh,
        compiler_params=pltpu.CompilerParams(needs_layout_passes=False,
                                             use_tc_tiling_on_sc=True),
        scratch_types=[
            pltpu.VMEM((BPW,), jnp.int32),
            pltpu.VMEM((BPW,), jnp.int32),
            pltpu.VMEM((BPW,), jnp.int32),
            pltpu.VMEM((BPW,), jnp.int32),
            pltpu.VMEM((BPW,), jnp.int32),
            pltpu.VMEM((BPW,), jnp.int32),
            pltpu.VMEM((QUARTER, 128), jnp.float32),
            pltpu.VMEM((QUARTER, 128), jnp.float32),
            pltpu.VMEM((QUARTER, 128), jnp.float32),
            pltpu.VMEM((QUARTER, 128), jnp.float32),
            pltpu.VMEM((QUARTER, 128), jnp.float32),
            pltpu.VMEM((QUARTER, 128), jnp.float32),
            pltpu.VMEM((BPW,), jnp.float32),
            pltpu.SemaphoreType.DMA,
            pltpu.SemaphoreType.DMA,
        ],
    )
    return f(users_m2, items_m2, users, pos, neg)


def _cos_body(n, chunk, au, aum, ai, aim, out, acc):
    i = pl.program_id(0)

    @pl.when(i == 0)
    def _():
        acc[0] = 0.0

    valid = (i * chunk + lax.broadcasted_iota(jnp.int32, (chunk,), 0)) < n

    def pair(a_ref, b_ref):
        a = a_ref[...]
        b = b_ref[...]
        num = jnp.sum(a * b, axis=0)
        na = jnp.sum(a * a, axis=0)
        nb = jnp.sum(b * b, axis=0)
        ratio = num * lax.rsqrt(na * nb + 1e-20)
        return jnp.sum(jnp.where(valid, ratio, 0.0))

    acc[0] += pair(au, aum) + pair(ai, aim)

    @pl.when(i == pl.num_programs(0) - 1)
    def _():
        out[0, 0] = acc[0]


def _tc_cos(ut, umt, it_, imt):
    d, n = ut.shape
    chunk = 8192
    grid = (n + chunk - 1) // chunk
    tbl_spec = pl.BlockSpec((d, chunk), lambda i: (0, i))
    return pl.pallas_call(
        functools.partial(_cos_body, n, chunk),
        grid=(grid,),
        in_specs=[tbl_spec, tbl_spec, tbl_spec, tbl_spec],
        out_specs=pl.BlockSpec(memory_space=pltpu.SMEM),
        out_shape=jax.ShapeDtypeStruct((1, 1), jnp.float32),
        scratch_shapes=[pltpu.SMEM((1,), jnp.float32)],
        compiler_params=pltpu.CompilerParams(
            dimension_semantics=("arbitrary",)),
    )(ut, umt, it_, imt)


def _final_body(n_rows, b, x, s, out):
    xx = x[...]
    bpr = jnp.sum(jnp.log(jax.nn.sigmoid(xx) + 1e-10))
    inv = 0.5 * (s[0, 0] / n_rows)
    mf = -(bpr / b)
    out[0, 0] = -inv + MASK_TAU * mf


def _tc_final(x2d, s, n_rows):
    b = x2d.shape[0] * x2d.shape[1]
    return pl.pallas_call(
        functools.partial(_final_body, float(n_rows), float(b)),
        in_specs=[pl.BlockSpec(x2d.shape, lambda: (0, 0)),
                  pl.BlockSpec(memory_space=pltpu.SMEM)],
        out_specs=pl.BlockSpec(memory_space=pltpu.SMEM),
        out_shape=jax.ShapeDtypeStruct((1, 1), jnp.float32),
    )(x2d, s)


def kernel(all_users, all_items, all_users_m, all_items_m, users, pos_items, neg_items):
    n = all_users.shape[0]
    x = _sc_scores(all_users_m.reshape(n // 2, 128),
                   all_items_m.reshape(n // 2, 128),
                   users, pos_items, neg_items)
    s = _tc_cos(jnp.swapaxes(all_users, 0, 1), jnp.swapaxes(all_users_m, 0, 1),
                jnp.swapaxes(all_items, 0, 1), jnp.swapaxes(all_items_m, 0, 1))
    loss = _tc_final(x.reshape(128, 128), s, n)
    return loss[0, 0]


# speedup vs baseline: 1.4170x; 1.4170x over previous
import functools

import jax
import jax.numpy as jnp
from jax import lax
from jax.experimental import pallas as pl
from jax.experimental.pallas import tpu as pltpu
from jax.experimental.pallas import tpu_sc as plsc

MASK_TAU = 0.5
L = 16
NC, NS = 2, 16
NW = NC * NS
BPW = 512
QUARTER = 128


def _sc_body(users_m, items_m, users, pos, neg, out,
             idx_u, idx_p, idx_n, bu0, bp0, bn0, bu1, bp1, bn1,
             x_v, sem0, sem1):
    wid = lax.axis_index("s") * NC + lax.axis_index("c")
    base = wid * BPW
    pltpu.sync_copy(users.at[pl.ds(base, BPW)], idx_u)
    pltpu.sync_copy(pos.at[pl.ds(base, BPW)], idx_p)
    pltpu.sync_copy(neg.at[pl.ds(base, BPW)], idx_n)

    bufs = [(bu0, bp0, bn0, sem0), (bu1, bp1, bn1, sem1)]

    def fire(q):
        bu, bp, bn, sem = bufs[q % 2]
        sl = pl.ds(q * QUARTER, QUARTER)
        return [
            pltpu.async_copy(users_m.at[idx_u.at[sl]], bu, sem),
            pltpu.async_copy(items_m.at[idx_p.at[sl]], bp, sem),
            pltpu.async_copy(items_m.at[idx_n.at[sl]], bn, sem),
        ]

    lanes = lax.iota(jnp.int32, L)
    cb = [jnp.bitwise_and(lanes + k, L - 1) for k in range(L)]

    n_q = BPW // QUARTER
    pend = fire(0)
    for q in range(n_q):
        nxt = fire(q + 1) if q + 1 < n_q else []
        for cp in pend:
            cp.wait()
        pend = nxt
        bu, bp, bn, _ = bufs[q % 2]

        @pl.loop(0, QUARTER // L)
        def _blk(b0):
            row = b0 * L + lanes
            acc = [jnp.zeros((L,), jnp.float32) for _ in range(4)]
            for g in range(4):
                for k in range(L):
                    col = cb[k] + (16 * g)
                    u = plsc.load_gather(bu, [row, col])
                    p = plsc.load_gather(bp, [row, col])
                    n = plsc.load_gather(bn, [row, col])
                    acc[k % 4] = acc[k % 4] + u * (p - n)
            x_v[pl.ds(q * QUARTER + b0 * L, L)] = (
                (acc[0] + acc[1]) + (acc[2] + acc[3]))

    pltpu.sync_copy(x_v, out.at[pl.ds(base, BPW)])


def _sc_scores(users_m_pad, items_m_pad, users, pos, neg):
    b = users.shape[0]
    mesh = plsc.VectorSubcoreMesh(core_axis_name="c", subcore_axis_name="s",
                                  num_cores=NC, num_subcores=NS)
    f = pl.kernel(
        _sc_body,
        out_type=jax.ShapeDtypeStruct((b,), jnp.float32),
        mesh=mesh,
        compiler_params=pltpu.CompilerParams(needs_layout_passes=False,
                                             use_tc_tiling_on_sc=True),
        scratch_types=[
            pltpu.VMEM((BPW,), jnp.int32),
            pltpu.VMEM((BPW,), jnp.int32),
            pltpu.VMEM((BPW,), jnp.int32),
            pltpu.VMEM((QUARTER, 128), jnp.float32),
            pltpu.VMEM((QUARTER, 128), jnp.float32),
            pltpu.VMEM((QUARTER, 128), jnp.float32),
            pltpu.VMEM((QUARTER, 128), jnp.float32),
            pltpu.VMEM((QUARTER, 128), jnp.float32),
            pltpu.VMEM((QUARTER, 128), jnp.float32),
            pltpu.VMEM((BPW,), jnp.float32),
            pltpu.SemaphoreType.DMA,
            pltpu.SemaphoreType.DMA,
        ],
    )
    return f(users_m_pad, items_m_pad, users, pos, neg)


def _cos_body(n, chunk, au, aum, ai, aim, out, ump, imp, acc):
    i = pl.program_id(0)

    @pl.when(i == 0)
    def _():
        acc[0] = 0.0

    valid = (i * chunk + lax.broadcasted_iota(jnp.int32, (chunk,), 0)) < n

    def pair(a_ref, b_ref):
        a = a_ref[...]
        b = b_ref[...]
        num = jnp.sum(a * b, axis=0)
        na = jnp.sum(a * a, axis=0)
        nb = jnp.sum(b * b, axis=0)
        ratio = num * lax.rsqrt(na * nb + 1e-20)
        return jnp.sum(jnp.where(valid, ratio, 0.0))

    acc[0] += pair(au, aum) + pair(ai, aim)

    eye = jnp.eye(64, dtype=jnp.float32)
    dims = (((0,), (0,)), ((), ()))
    zer = jnp.zeros((chunk, 64), jnp.float32)
    tu = lax.dot_general(aum[...], eye, dims,
                         preferred_element_type=jnp.float32)
    ump[...] = jnp.concatenate([tu, zer], axis=1)
    ti = lax.dot_general(aim[...], eye, dims,
                         preferred_element_type=jnp.float32)
    imp[...] = jnp.concatenate([ti, zer], axis=1)

    @pl.when(i == pl.num_programs(0) - 1)
    def _():
        out[0, 0] = acc[0]


def _tc_cos(ut, umt, it_, imt):
    d, n = ut.shape
    chunk = 4096
    grid = (n + chunk - 1) // chunk
    tbl_spec = pl.BlockSpec((d, chunk), lambda i: (0, i))
    pad_spec = pl.BlockSpec((chunk, 128), lambda i: (i, 0))
    return pl.pallas_call(
        functools.partial(_cos_body, n, chunk),
        grid=(grid,),
        in_specs=[tbl_spec, tbl_spec, tbl_spec, tbl_spec],
        out_specs=[pl.BlockSpec(memory_space=pltpu.SMEM),
                   pad_spec, pad_spec],
        out_shape=[jax.ShapeDtypeStruct((1, 1), jnp.float32),
                   jax.ShapeDtypeStruct((n, 128), jnp.float32),
                   jax.ShapeDtypeStruct((n, 128), jnp.float32)],
        scratch_shapes=[pltpu.SMEM((1,), jnp.float32)],
        compiler_params=pltpu.CompilerParams(
            dimension_semantics=("arbitrary",)),
    )(ut, umt, it_, imt)


def _final_body(n_rows, b, x, s, out):
    xx = x[...]
    bpr = jnp.sum(jnp.log(jax.nn.sigmoid(xx) + 1e-10))
    inv = 0.5 * (s[0, 0] / n_rows)
    mf = -(bpr / b)
    out[0, 0] = -inv + MASK_TAU * mf


def _tc_final(x2d, s, n_rows):
    b = x2d.shape[0] * x2d.shape[1]
    return pl.pallas_call(
        functools.partial(_final_body, float(n_rows), float(b)),
        in_specs=[pl.BlockSpec(x2d.shape, lambda: (0, 0)),
                  pl.BlockSpec(memory_space=pltpu.SMEM)],
        out_specs=pl.BlockSpec(memory_space=pltpu.SMEM),
        out_shape=jax.ShapeDtypeStruct((1, 1), jnp.float32),
    )(x2d, s)


def kernel(all_users, all_items, all_users_m, all_items_m, users, pos_items, neg_items):
    n = all_users.shape[0]
    s, um_pad, im_pad = _tc_cos(
        jnp.swapaxes(all_users, 0, 1), jnp.swapaxes(all_users_m, 0, 1),
        jnp.swapaxes(all_items, 0, 1), jnp.swapaxes(all_items_m, 0, 1))
    x = _sc_scores(um_pad, im_pad, users, pos_items, neg_items)
    loss = _tc_final(x.reshape(128, 128), s, n)
    return loss[0, 0]


# speedup vs baseline: 1.5584x; 1.0998x over previous
import functools

import jax
import jax.numpy as jnp
from jax import lax
from jax.experimental import pallas as pl
from jax.experimental.pallas import tpu as pltpu
from jax.experimental.pallas import tpu_sc as plsc

MASK_TAU = 0.5
L = 16
NC, NS = 2, 16
NW = NC * NS
BPW = 512
QUARTER = 128


def _sc_body(users_m, items_m, users, pos, neg, out,
             idx_u, idx_p, idx_n, bu0, bp0, bn0, bu1, bp1, bn1,
             x_v, sem0, sem1):
    wid = lax.axis_index("s") * NC + lax.axis_index("c")
    base = wid * BPW
    pltpu.sync_copy(users.at[pl.ds(base, BPW)], idx_u)
    pltpu.sync_copy(pos.at[pl.ds(base, BPW)], idx_p)
    pltpu.sync_copy(neg.at[pl.ds(base, BPW)], idx_n)

    bufs = [(bu0, bp0, bn0, sem0), (bu1, bp1, bn1, sem1)]

    def fire(q):
        bu, bp, bn, sem = bufs[q % 2]
        sl = pl.ds(q * QUARTER, QUARTER)
        return [
            pltpu.async_copy(users_m.at[idx_u.at[sl]], bu, sem),
            pltpu.async_copy(items_m.at[idx_p.at[sl]], bp, sem),
            pltpu.async_copy(items_m.at[idx_n.at[sl]], bn, sem),
        ]

    lanes = lax.iota(jnp.int32, L)
    cb = [jnp.bitwise_and(lanes + k, L - 1) for k in range(L)]

    n_q = BPW // QUARTER
    pend = fire(0)
    for q in range(n_q):
        nxt = fire(q + 1) if q + 1 < n_q else []
        for cp in pend:
            cp.wait()
        pend = nxt
        bu, bp, bn, _ = bufs[q % 2]

        @pl.loop(0, QUARTER // L)
        def _blk(b0):
            row = b0 * L + lanes
            acc = [jnp.zeros((L,), jnp.float32) for _ in range(4)]
            for g in range(4):
                for k in range(L):
                    col = cb[k] + (16 * g)
                    u = plsc.load_gather(bu, [row, col])
                    p = plsc.load_gather(bp, [row, col])
                    n = plsc.load_gather(bn, [row, col])
                    acc[k % 4] = acc[k % 4] + u * (p - n)
            x_v[pl.ds(q * QUARTER + b0 * L, L)] = (
                (acc[0] + acc[1]) + (acc[2] + acc[3]))

    pltpu.sync_copy(x_v, out.at[pl.ds(base, BPW)])


def _sc_scores(users_m_pad, items_m_pad, users, pos, neg):
    b = users.shape[0]
    mesh = plsc.VectorSubcoreMesh(core_axis_name="c", subcore_axis_name="s",
                                  num_cores=NC, num_subcores=NS)
    f = pl.kernel(
        _sc_body,
        out_type=jax.ShapeDtypeStruct((b,), jnp.float32),
        mesh=mesh,
        compiler_params=pltpu.CompilerParams(needs_layout_passes=False,
                                             use_tc_tiling_on_sc=True),
        scratch_types=[
            pltpu.VMEM((BPW,), jnp.int32),
            pltpu.VMEM((BPW,), jnp.int32),
            pltpu.VMEM((BPW,), jnp.int32),
            pltpu.VMEM((QUARTER, 128), jnp.float32),
            pltpu.VMEM((QUARTER, 128), jnp.float32),
            pltpu.VMEM((QUARTER, 128), jnp.float32),
            pltpu.VMEM((QUARTER, 128), jnp.float32),
            pltpu.VMEM((QUARTER, 128), jnp.float32),
            pltpu.VMEM((QUARTER, 128), jnp.float32),
            pltpu.VMEM((BPW,), jnp.float32),
            pltpu.SemaphoreType.DMA,
            pltpu.SemaphoreType.DMA,
        ],
    )
    return f(users_m_pad, items_m_pad, users, pos, neg)


def _emit_body(chunk, aum, aim, ump, imp):
    eye = jnp.eye(64, dtype=jnp.float32)
    dims = (((0,), (0,)), ((), ()))
    zer = jnp.zeros((chunk, 64), jnp.float32)
    tu = lax.dot_general(aum[...], eye, dims,
                         preferred_element_type=jnp.float32)
    ump[...] = jnp.concatenate([tu, zer], axis=1)
    ti = lax.dot_general(aim[...], eye, dims,
                         preferred_element_type=jnp.float32)
    imp[...] = jnp.concatenate([ti, zer], axis=1)


def _tc_emit(umt, imt):
    d, n = umt.shape
    chunk = 8192
    grid = (n + chunk - 1) // chunk
    tbl_spec = pl.BlockSpec((d, chunk), lambda i: (0, i))
    pad_spec = pl.BlockSpec((chunk, 128), lambda i: (i, 0))
    return pl.pallas_call(
        functools.partial(_emit_body, chunk),
        grid=(grid,),
        in_specs=[tbl_spec, tbl_spec],
        out_specs=[pad_spec, pad_spec],
        out_shape=[jax.ShapeDtypeStruct((n, 128), jnp.float32),
                   jax.ShapeDtypeStruct((n, 128), jnp.float32)],
        compiler_params=pltpu.CompilerParams(
            dimension_semantics=("arbitrary",)),
    )(umt, imt)


def _cos_body(n, chunk, au, aum, ai, aim, out, acc):
    i = pl.program_id(0)

    @pl.when(i == 0)
    def _():
        acc[0] = 0.0

    valid = (i * chunk + lax.broadcasted_iota(jnp.int32, (chunk,), 0)) < n

    def pair(a_ref, b_ref):
        a = a_ref[...]
        b = b_ref[...]
        num = jnp.sum(a * b, axis=0)
        na = jnp.sum(a * a, axis=0)
        nb = jnp.sum(b * b, axis=0)
        ratio = num * lax.rsqrt(na * nb + 1e-20)
        return jnp.sum(jnp.where(valid, ratio, 0.0))

    acc[0] += pair(au, aum) + pair(ai, aim)

    @pl.when(i == pl.num_programs(0) - 1)
    def _():
        out[0, 0] = acc[0]


def _tc_cos(ut, umt, it_, imt):
    d, n = ut.shape
    chunk = 8192
    grid = (n + chunk - 1) // chunk
    tbl_spec = pl.BlockSpec((d, chunk), lambda i: (0, i))
    return pl.pallas_call(
        functools.partial(_cos_body, n, chunk),
        grid=(grid,),
        in_specs=[tbl_spec, tbl_spec, tbl_spec, tbl_spec],
        out_specs=pl.BlockSpec(memory_space=pltpu.SMEM),
        out_shape=jax.ShapeDtypeStruct((1, 1), jnp.float32),
        scratch_shapes=[pltpu.SMEM((1,), jnp.float32)],
        compiler_params=pltpu.CompilerParams(
            dimension_semantics=("arbitrary",)),
    )(ut, umt, it_, imt)


def _final_body(n_rows, b, x, s, out):
    xx = x[...]
    bpr = jnp.sum(jnp.log(jax.nn.sigmoid(xx) + 1e-10))
    inv = 0.5 * (s[0, 0] / n_rows)
    mf = -(bpr / b)
    out[0, 0] = -inv + MASK_TAU * mf


def _tc_final(x2d, s, n_rows):
    b = x2d.shape[0] * x2d.shape[1]
    return pl.pallas_call(
        functools.partial(_final_body, float(n_rows), float(b)),
        in_specs=[pl.BlockSpec(x2d.shape, lambda: (0, 0)),
                  pl.BlockSpec(memory_space=pltpu.SMEM)],
        out_specs=pl.BlockSpec(memory_space=pltpu.SMEM),
        out_shape=jax.ShapeDtypeStruct((1, 1), jnp.float32),
    )(x2d, s)


def kernel(all_users, all_items, all_users_m, all_items_m, users, pos_items, neg_items):
    n = all_users.shape[0]
    umt = jnp.swapaxes(all_users_m, 0, 1)
    imt = jnp.swapaxes(all_items_m, 0, 1)
    um_pad, im_pad = _tc_emit(umt, imt)
    x = _sc_scores(um_pad, im_pad, users, pos_items, neg_items)
    s = _tc_cos(jnp.swapaxes(all_users, 0, 1), umt,
                jnp.swapaxes(all_items, 0, 1), imt)
    loss = _tc_final(x.reshape(128, 128), s, n)
    return loss[0, 0]


# speedup vs baseline: 1.5756x; 1.0110x over previous
import functools

import jax
import jax.numpy as jnp
from jax import lax
from jax.experimental import pallas as pl
from jax.experimental.pallas import tpu as pltpu
from jax.experimental.pallas import tpu_sc as plsc

MASK_TAU = 0.5
L = 16
NC, NS = 2, 16
NW = NC * NS
BPW = 512
QUARTER = 128


def _sc_body(comb, users, pos, neg, out,
             idx_u, idx_p, idx_n, bu0, bp0, bn0, bu1, bp1, bn1,
             x_v, sem0, sem1):
    wid = lax.axis_index("s") * NC + lax.axis_index("c")
    base = wid * BPW
    pltpu.sync_copy(users.at[pl.ds(base, BPW)], idx_u)
    pltpu.sync_copy(pos.at[pl.ds(base, BPW)], idx_p)
    pltpu.sync_copy(neg.at[pl.ds(base, BPW)], idx_n)

    bufs = [(bu0, bp0, bn0, sem0), (bu1, bp1, bn1, sem1)]

    def fire(q):
        bu, bp, bn, sem = bufs[q % 2]
        sl = pl.ds(q * QUARTER, QUARTER)
        return [
            pltpu.async_copy(comb.at[idx_u.at[sl]], bu, sem),
            pltpu.async_copy(comb.at[idx_p.at[sl]], bp, sem),
            pltpu.async_copy(comb.at[idx_n.at[sl]], bn, sem),
        ]

    lanes = lax.iota(jnp.int32, L)
    cb = [jnp.bitwise_and(lanes + k, L - 1) for k in range(L)]

    n_q = BPW // QUARTER
    pend = fire(0)
    for q in range(n_q):
        nxt = fire(q + 1) if q + 1 < n_q else []
        for cp in pend:
            cp.wait()
        pend = nxt
        bu, bp, bn, _ = bufs[q % 2]

        @pl.loop(0, QUARTER // L)
        def _blk(b0):
            row = b0 * L + lanes
            acc = [jnp.zeros((L,), jnp.float32) for _ in range(4)]
            for g in range(4):
                for k in range(L):
                    col = cb[k] + (16 * g)
                    u = plsc.load_gather(bu, [row, col])
                    p = plsc.load_gather(bp, [row, col + 64])
                    n = plsc.load_gather(bn, [row, col + 64])
                    acc[k % 4] = acc[k % 4] + u * (p - n)
            x_v[pl.ds(q * QUARTER + b0 * L, L)] = (
                (acc[0] + acc[1]) + (acc[2] + acc[3]))

    pltpu.sync_copy(x_v, out.at[pl.ds(base, BPW)])


def _sc_scores(comb, users, pos, neg):
    b = users.shape[0]
    mesh = plsc.VectorSubcoreMesh(core_axis_name="c", subcore_axis_name="s",
                                  num_cores=NC, num_subcores=NS)
    f = pl.kernel(
        _sc_body,
        out_type=jax.ShapeDtypeStruct((b,), jnp.float32),
        mesh=mesh,
        compiler_params=pltpu.CompilerParams(needs_layout_passes=False,
                                             use_tc_tiling_on_sc=True),
        scratch_types=[
            pltpu.VMEM((BPW,), jnp.int32),
            pltpu.VMEM((BPW,), jnp.int32),
            pltpu.VMEM((BPW,), jnp.int32),
            pltpu.VMEM((QUARTER, 128), jnp.float32),
            pltpu.VMEM((QUARTER, 128), jnp.float32),
            pltpu.VMEM((QUARTER, 128), jnp.float32),
            pltpu.VMEM((QUARTER, 128), jnp.float32),
            pltpu.VMEM((QUARTER, 128), jnp.float32),
            pltpu.VMEM((QUARTER, 128), jnp.float32),
            pltpu.VMEM((BPW,), jnp.float32),
            pltpu.SemaphoreType.DMA,
            pltpu.SemaphoreType.DMA,
        ],
    )
    return f(comb, users, pos, neg)


def _emit_body(chunk, aum, aim, cmb):
    eye = jnp.eye(64, dtype=jnp.float32)
    dims = (((0,), (0,)), ((), ()))
    tu = lax.dot_general(aum[...], eye, dims,
                         preferred_element_type=jnp.float32)
    ti = lax.dot_general(aim[...], eye, dims,
                         preferred_element_type=jnp.float32)
    cmb[...] = jnp.concatenate([tu, ti], axis=1)


def _tc_emit(umt, imt):
    d, n = umt.shape
    chunk = 8192
    grid = (n + chunk - 1) // chunk
    tbl_spec = pl.BlockSpec((d, chunk), lambda i: (0, i))
    pad_spec = pl.BlockSpec((chunk, 128), lambda i: (i, 0))
    return pl.pallas_call(
        functools.partial(_emit_body, chunk),
        grid=(grid,),
        in_specs=[tbl_spec, tbl_spec],
        out_specs=pad_spec,
        out_shape=jax.ShapeDtypeStruct((n, 128), jnp.float32),
        compiler_params=pltpu.CompilerParams(
            dimension_semantics=("arbitrary",)),
    )(umt, imt)


def _cos_body(n, chunk, au, aum, ai, aim, out, acc):
    i = pl.program_id(0)

    @pl.when(i == 0)
    def _():
        acc[0] = 0.0

    valid = (i * chunk + lax.broadcasted_iota(jnp.int32, (chunk,), 0)) < n

    def pair(a_ref, b_ref):
        a = a_ref[...]
        b = b_ref[...]
        num = jnp.sum(a * b, axis=0)
        na = jnp.sum(a * a, axis=0)
        nb = jnp.sum(b * b, axis=0)
        ratio = num * lax.rsqrt(na * nb + 1e-20)
        return jnp.sum(jnp.where(valid, ratio, 0.0))

    acc[0] += pair(au, aum) + pair(ai, aim)

    @pl.when(i == pl.num_programs(0) - 1)
    def _():
        out[0, 0] = acc[0]


def _tc_cos(ut, umt, it_, imt):
    d, n = ut.shape
    chunk = 8192
    grid = (n + chunk - 1) // chunk
    tbl_spec = pl.BlockSpec((d, chunk), lambda i: (0, i))
    return pl.pallas_call(
        functools.partial(_cos_body, n, chunk),
        grid=(grid,),
        in_specs=[tbl_spec, tbl_spec, tbl_spec, tbl_spec],
        out_specs=pl.BlockSpec(memory_space=pltpu.SMEM),
        out_shape=jax.ShapeDtypeStruct((1, 1), jnp.float32),
        scratch_shapes=[pltpu.SMEM((1,), jnp.float32)],
        compiler_params=pltpu.CompilerParams(
            dimension_semantics=("arbitrary",)),
    )(ut, umt, it_, imt)


def _final_body(n_rows, b, x, s, out):
    xx = x[...]
    bpr = jnp.sum(jnp.log(jax.nn.sigmoid(xx) + 1e-10))
    inv = 0.5 * (s[0, 0] / n_rows)
    mf = -(bpr / b)
    out[0, 0] = -inv + MASK_TAU * mf


def _tc_final(x2d, s, n_rows):
    b = x2d.shape[0] * x2d.shape[1]
    return pl.pallas_call(
        functools.partial(_final_body, float(n_rows), float(b)),
        in_specs=[pl.BlockSpec(x2d.shape, lambda: (0, 0)),
                  pl.BlockSpec(memory_space=pltpu.SMEM)],
        out_specs=pl.BlockSpec(memory_space=pltpu.SMEM),
        out_shape=jax.ShapeDtypeStruct((1, 1), jnp.float32),
    )(x2d, s)


def kernel(all_users, all_items, all_users_m, all_items_m, users, pos_items, neg_items):
    n = all_users.shape[0]
    umt = jnp.swapaxes(all_users_m, 0, 1)
    imt = jnp.swapaxes(all_items_m, 0, 1)
    comb = _tc_emit(umt, imt)
    x = _sc_scores(comb, users, pos_items, neg_items)
    s = _tc_cos(jnp.swapaxes(all_users, 0, 1), umt,
                jnp.swapaxes(all_items, 0, 1), imt)
    loss = _tc_final(x.reshape(128, 128), s, n)
    return loss[0, 0]


# speedup vs baseline: 1.6856x; 1.0698x over previous
import functools

import jax
import jax.numpy as jnp
from jax import lax
from jax.experimental import pallas as pl
from jax.experimental.pallas import tpu as pltpu
from jax.experimental.pallas import tpu_sc as plsc

MASK_TAU = 0.5
L = 16
NC, NS = 2, 16
NW = NC * NS
BPW = 512
QUARTER = 128


def _sc_body(comb, users, pos, neg, out,
             idx_u, idx_p, idx_n, bu0, bp0, bn0, bu1, bp1, bn1,
             x_v, sem0, sem1):
    wid = lax.axis_index("s") * NC + lax.axis_index("c")
    base = wid * BPW
    pltpu.sync_copy(users.at[pl.ds(base, BPW)], idx_u)
    pltpu.sync_copy(pos.at[pl.ds(base, BPW)], idx_p)
    pltpu.sync_copy(neg.at[pl.ds(base, BPW)], idx_n)

    bufs = [(bu0, bp0, bn0, sem0), (bu1, bp1, bn1, sem1)]

    def fire(q):
        bu, bp, bn, sem = bufs[q % 2]
        sl = pl.ds(q * QUARTER, QUARTER)
        return [
            pltpu.async_copy(comb.at[idx_u.at[sl]], bu, sem),
            pltpu.async_copy(comb.at[idx_p.at[sl]], bp, sem),
            pltpu.async_copy(comb.at[idx_n.at[sl]], bn, sem),
        ]

    lanes = lax.iota(jnp.int32, L)
    cb = [jnp.bitwise_and(lanes + k, L - 1) for k in range(L)]

    n_q = BPW // QUARTER
    pend = fire(0)
    for q in range(n_q):
        nxt = fire(q + 1) if q + 1 < n_q else []
        for cp in pend:
            cp.wait()
        pend = nxt
        bu, bp, bn, _ = bufs[q % 2]

        @pl.loop(0, QUARTER // L)
        def _blk(b0):
            row = b0 * L + lanes
            acc = [jnp.zeros((L,), jnp.float32) for _ in range(4)]
            for g in range(4):
                for k in range(L):
                    col = cb[k] + (16 * g)
                    u = plsc.load_gather(bu, [row, col])
                    p = plsc.load_gather(bp, [row, col + 64])
                    n = plsc.load_gather(bn, [row, col + 64])
                    acc[k % 4] = acc[k % 4] + u * (p - n)
            x_v[pl.ds(q * QUARTER + b0 * L, L)] = (
                (acc[0] + acc[1]) + (acc[2] + acc[3]))

    pltpu.sync_copy(x_v, out.at[pl.ds(base, BPW)])


def _sc_scores(comb, users, pos, neg):
    b = users.shape[0]
    mesh = plsc.VectorSubcoreMesh(core_axis_name="c", subcore_axis_name="s",
                                  num_cores=NC, num_subcores=NS)
    f = pl.kernel(
        _sc_body,
        out_type=jax.ShapeDtypeStruct((b,), jnp.float32),
        mesh=mesh,
        compiler_params=pltpu.CompilerParams(needs_layout_passes=False,
                                             use_tc_tiling_on_sc=True),
        scratch_types=[
            pltpu.VMEM((BPW,), jnp.int32),
            pltpu.VMEM((BPW,), jnp.int32),
            pltpu.VMEM((BPW,), jnp.int32),
            pltpu.VMEM((QUARTER, 128), jnp.float32),
            pltpu.VMEM((QUARTER, 128), jnp.float32),
            pltpu.VMEM((QUARTER, 128), jnp.float32),
            pltpu.VMEM((QUARTER, 128), jnp.float32),
            pltpu.VMEM((QUARTER, 128), jnp.float32),
            pltpu.VMEM((QUARTER, 128), jnp.float32),
            pltpu.VMEM((BPW,), jnp.float32),
            pltpu.SemaphoreType.DMA,
            pltpu.SemaphoreType.DMA,
        ],
    )
    return f(comb, users, pos, neg)


def _emit_body(chunk, aum, aim, cmb):
    r = lax.broadcasted_iota(jnp.int32, (64, 128), 0)
    c = lax.broadcasted_iota(jnp.int32, (64, 128), 1)
    eye_lo = jnp.where(r == c, 1.0, 0.0)
    eye_hi = jnp.where(r + 64 == c, 1.0, 0.0)
    dims = (((0,), (0,)), ((), ()))
    tu = lax.dot_general(aum[...], eye_lo, dims,
                         preferred_element_type=jnp.float32)
    ti = lax.dot_general(aim[...], eye_hi, dims,
                         preferred_element_type=jnp.float32)
    cmb[...] = tu + ti


def _tc_emit(umt, imt):
    d, n = umt.shape
    chunk = 8192
    grid = (n + chunk - 1) // chunk
    tbl_spec = pl.BlockSpec((d, chunk), lambda i: (0, i))
    pad_spec = pl.BlockSpec((chunk, 128), lambda i: (i, 0))
    return pl.pallas_call(
        functools.partial(_emit_body, chunk),
        grid=(grid,),
        in_specs=[tbl_spec, tbl_spec],
        out_specs=pad_spec,
        out_shape=jax.ShapeDtypeStruct((n, 128), jnp.float32),
        compiler_params=pltpu.CompilerParams(
            dimension_semantics=("arbitrary",)),
    )(umt, imt)


def _cos_body(n, chunk, au, aum, ai, aim, out, acc):
    i = pl.program_id(0)

    @pl.when(i == 0)
    def _():
        acc[0] = 0.0

    valid = (i * chunk + lax.broadcasted_iota(jnp.int32, (chunk,), 0)) < n

    def pair(a_ref, b_ref):
        a = a_ref[...]
        b = b_ref[...]
        num = jnp.sum(a * b, axis=0)
        na = jnp.sum(a * a, axis=0)
        nb = jnp.sum(b * b, axis=0)
        ratio = num * lax.rsqrt(na * nb + 1e-20)
        return jnp.sum(jnp.where(valid, ratio, 0.0))

    acc[0] += pair(au, aum) + pair(ai, aim)

    @pl.when(i == pl.num_programs(0) - 1)
    def _():
        out[0, 0] = acc[0]


def _tc_cos(ut, umt, it_, imt):
    d, n = ut.shape
    chunk = 8192
    grid = (n + chunk - 1) // chunk
    tbl_spec = pl.BlockSpec((d, chunk), lambda i: (0, i))
    return pl.pallas_call(
        functools.partial(_cos_body, n, chunk),
        grid=(grid,),
        in_specs=[tbl_spec, tbl_spec, tbl_spec, tbl_spec],
        out_specs=pl.BlockSpec(memory_space=pltpu.SMEM),
        out_shape=jax.ShapeDtypeStruct((1, 1), jnp.float32),
        scratch_shapes=[pltpu.SMEM((1,), jnp.float32)],
        compiler_params=pltpu.CompilerParams(
            dimension_semantics=("arbitrary",)),
    )(ut, umt, it_, imt)


def _final_body(n_rows, b, x, s, out):
    xx = x[...]
    bpr = jnp.sum(jnp.log(jax.nn.sigmoid(xx) + 1e-10))
    inv = 0.5 * (s[0, 0] / n_rows)
    mf = -(bpr / b)
    out[0, 0] = -inv + MASK_TAU * mf


def _tc_final(x2d, s, n_rows):
    b = x2d.shape[0] * x2d.shape[1]
    return pl.pallas_call(
        functools.partial(_final_body, float(n_rows), float(b)),
        in_specs=[pl.BlockSpec(x2d.shape, lambda: (0, 0)),
                  pl.BlockSpec(memory_space=pltpu.SMEM)],
        out_specs=pl.BlockSpec(memory_space=pltpu.SMEM),
        out_shape=jax.ShapeDtypeStruct((1, 1), jnp.float32),
    )(x2d, s)


def kernel(all_users, all_items, all_users_m, all_items_m, users, pos_items, neg_items):
    n = all_users.shape[0]
    umt = jnp.swapaxes(all_users_m, 0, 1)
    imt = jnp.swapaxes(all_items_m, 0, 1)
    comb = _tc_emit(umt, imt)
    x = _sc_scores(comb, users, pos_items, neg_items)
    s = _tc_cos(jnp.swapaxes(all_users, 0, 1), umt,
                jnp.swapaxes(all_items, 0, 1), imt)
    loss = _tc_final(x.reshape(128, 128), s, n)
    return loss[0, 0]
